# baseline (speedup 1.0000x reference)
import jax
import jax.numpy as jnp
from jax.experimental import pallas as pl
from jax.experimental.pallas import tpu as pltpu

_DNT = (((1,), (1,)), ((), ()))
_DN0 = (((0,), (0,)), ((), ()))


def _mushin_body(inp_ref, inc_ref, wattr_ref, wenc_ref, benc_ref, battr_ref,
                 wconv_ref, att_ref, bconv_ref, wout_ref, bout_ref, out_ref,
                 xl0_s, xl1_s, xl2_s, inc_s, he_s):
    f32 = jnp.float32
    heads, two_c = att_ref.shape
    c = two_c // 2
    blk = inp_ref.shape[0]
    i = pl.program_id(0)
    k = pl.num_programs(0)

    x_b = jnp.dot(inp_ref[...], wenc_ref[...], preferred_element_type=f32)
    x_b = jnp.maximum(x_b + benc_ref[...], 0.0)
    row = pl.ds(i * blk, blk)
    for h, xl_s in enumerate((xl0_s, xl1_s, xl2_s)):
        xl_s[row, :] = jnp.dot(x_b, wconv_ref[:, h * c:(h + 1) * c],
                               preferred_element_type=f32)
    inc_s[row, :] = inc_ref[...]
    he_part = jax.lax.dot_general(inc_ref[...], wattr_ref[...], _DN0,
                                  preferred_element_type=f32)

    @pl.when(i == 0)
    def _init():
        he_s[...] = he_part

    @pl.when(i > 0)
    def _acc():
        he_s[...] += he_part

    @pl.when(i == k - 1)
    def _finish():
        incT = jnp.transpose(inc_s[...])
        maskT = incT > 0.0
        he = he_s[...] + battr_ref[...]

        deg_n = jnp.sum(incT, axis=0, keepdims=True)
        inv_dn = jnp.where(deg_n > 0.0, 1.0 / deg_n, 0.0)
        deg_e = jnp.sum(incT, axis=1, keepdims=True)
        inv_de = jnp.where(deg_e > 0.0, 1.0 / deg_e, 0.0)

        res = None
        for h, xl_s in enumerate((xl0_s, xl1_s, xl2_s)):
            ai = att_ref[h:h + 1, :c]
            aj = att_ref[h:h + 1, c:]
            bc = bconv_ref[:, h * c:(h + 1) * c]
            wo = wout_ref[h * c:(h + 1) * c, :]

            xl = xl_s[...]
            ea = jnp.dot(he, wconv_ref[:, h * c:(h + 1) * c],
                         preferred_element_type=f32)
            a_i = jax.lax.dot_general(ai, xl, _DNT,
                                      preferred_element_type=f32)
            a_e = jax.lax.dot_general(ea, aj, _DNT,
                                      preferred_element_type=f32)
            logit = a_i + a_e
            logit = jnp.where(logit >= 0.0, logit, 0.2 * logit)
            lmask = jnp.where(maskT, logit, -1e30)
            amax = jnp.max(lmask, axis=0, keepdims=True)
            amax = jnp.where(amax > -1e29, amax, 0.0)
            ex = jnp.where(maskT, jnp.exp(logit - amax), 0.0)
            den = jnp.sum(ex, axis=0, keepdims=True)
            alphaT = ex / (den + 1e-16)

            out_e = inv_de * jnp.dot(alphaT, xl,
                                     preferred_element_type=f32)
            g = jax.lax.dot_general(incT, alphaT * inv_dn, _DNT,
                                    preferred_element_type=f32)
            hf = jnp.dot(g, out_e, preferred_element_type=f32)
            hf = hf + deg_e * bc
            part = jnp.dot(hf, wo, preferred_element_type=f32)
            res = part if res is None else res + part

        out_ref[...] = res + bout_ref[...]


def kernel(input_features, incidence_matrix, W_enc, b_enc, W_attr, b_attr,
           W_conv, att, b_conv, W_out, b_out):
    n, in_feat = input_features.shape
    m = incidence_matrix.shape[1]
    emb = W_enc.shape[1]
    heads = att.shape[1]
    conv = att.shape[2] // 2
    blk = 2000
    k = n // blk

    grid_spec = pltpu.PrefetchScalarGridSpec(
        num_scalar_prefetch=0,
        grid=(k,),
        in_specs=[
            pl.BlockSpec((blk, in_feat), lambda i: (i, 0)),
            pl.BlockSpec((blk, m), lambda i: (i, 0)),
            pl.BlockSpec((blk, emb), lambda i: (i, 0)),
            pl.BlockSpec((in_feat, emb), lambda i: (0, 0)),
            pl.BlockSpec((1, emb), lambda i: (0, 0)),
            pl.BlockSpec((1, emb), lambda i: (0, 0)),
            pl.BlockSpec((emb, heads * conv), lambda i: (0, 0)),
            pl.BlockSpec((heads, 2 * conv), lambda i: (0, 0)),
            pl.BlockSpec((1, heads * conv), lambda i: (0, 0)),
            pl.BlockSpec((heads * conv, 2), lambda i: (0, 0)),
            pl.BlockSpec((1, 2), lambda i: (0, 0)),
        ],
        out_specs=pl.BlockSpec((m, 2), lambda i: (0, 0)),
        scratch_shapes=[
            pltpu.VMEM((n, conv), jnp.float32),
            pltpu.VMEM((n, conv), jnp.float32),
            pltpu.VMEM((n, conv), jnp.float32),
            pltpu.VMEM((n, m), jnp.float32),
            pltpu.VMEM((m, emb), jnp.float32),
        ],
    )
    return pl.pallas_call(
        _mushin_body,
        grid_spec=grid_spec,
        out_shape=jax.ShapeDtypeStruct((m, 2), jnp.float32),
    )(input_features, incidence_matrix, W_attr, W_enc, b_enc.reshape(1, emb),
      b_attr.reshape(1, emb), W_conv, att.reshape(heads, -1),
      b_conv.reshape(1, -1), W_out, b_out.reshape(1, -1))
